# Initial kernel scaffold; baseline (speedup 1.0000x reference)
import jax
import jax.numpy as jnp
from jax.experimental import pallas as pl


def kernel(features, W_proj, b_proj, expert_emb, expert_features, trust, staleness_dt):
    raise NotImplementedError("write your pallas kernel here")


# speedup vs baseline: 1.4102x; 1.4102x over previous
import jax
import jax.numpy as jnp
from jax.experimental import pallas as pl
from jax.experimental.pallas import tpu as pltpu

FEATURE_DIM = 1024
HIDDEN_DIM = 256
NUM_EXPERTS = 64
TOP_K = 8
STALENESS_LAMBDA = 0.005
STALENESS_FLOOR = 0.1
N_TOKENS = 16384

BLOCK_B = 1024


def _router_kernel(f_ref, w_ref, b_ref, emb_ref, ef_ref, tr_ref, st_ref,
                   out_ref):
    f = f_ref[:]
    fb = f.astype(jnp.bfloat16)
    h = jnp.dot(fb, w_ref[:].astype(jnp.bfloat16),
                preferred_element_type=jnp.float32) + b_ref[:]
    logits = jax.lax.dot_general(
        h.astype(jnp.bfloat16), emb_ref[:].astype(jnp.bfloat16),
        (((1,), (1,)), ((), ())),
        preferred_element_type=jnp.float32)
    m = jnp.max(logits, axis=-1, keepdims=True)
    p = jnp.exp(logits - m)
    gate = p / jnp.sum(p, axis=-1, keepdims=True)

    ef = ef_ref[:]
    en = ef / (jnp.sqrt(jnp.sum(ef * ef, axis=1, keepdims=True)) + 1e-8)
    rnorm = jnp.sqrt(jnp.sum(f * f, axis=1, keepdims=True)) + 1e-8
    fn = f / rnorm
    sim = jnp.maximum(jax.lax.dot_general(
        fn.astype(jnp.bfloat16), en.astype(jnp.bfloat16),
        (((1,), (1,)), ((), ())),
        preferred_element_type=jnp.float32), 0.0)

    scores = ((gate * tr_ref[:]) * sim) * st_ref[:]

    t = jnp.full(scores.shape[:1] + (1,), jnp.inf, jnp.float32)
    removed = jnp.zeros(scores.shape[:1] + (1,), jnp.int32)
    kth = jnp.zeros(scores.shape[:1] + (1,), jnp.float32)
    for _ in range(TOP_K):
        cand = jnp.where(scores < t, scores, -1.0)
        mx = jnp.max(cand, axis=-1, keepdims=True)
        nrem = jnp.sum((scores >= mx).astype(jnp.int32), axis=-1,
                       keepdims=True)
        live = removed < TOP_K
        kth = jnp.where(live, mx, kth)
        t = jnp.where(live, mx, t)
        removed = jnp.where(live, nrem, removed)

    masked = jnp.where(scores >= kth, scores, 0.0)
    out_ref[:] = masked / (jnp.sum(masked, axis=-1, keepdims=True) + 1e-9)


@jax.jit
def kernel(features, W_proj, b_proj, expert_emb, expert_features, trust,
           staleness_dt):
    stale = jnp.maximum(jnp.exp(-STALENESS_LAMBDA * staleness_dt),
                        STALENESS_FLOOR)
    tr2 = trust.reshape(1, NUM_EXPERTS)
    st2 = stale.reshape(1, NUM_EXPERTS)
    b2 = b_proj.reshape(1, HIDDEN_DIM)
    n_blocks = N_TOKENS // BLOCK_B
    return pl.pallas_call(
        _router_kernel,
        grid=(n_blocks,),
        in_specs=[
            pl.BlockSpec((BLOCK_B, FEATURE_DIM), lambda i: (i, 0)),
            pl.BlockSpec((FEATURE_DIM, HIDDEN_DIM), lambda i: (0, 0)),
            pl.BlockSpec((1, HIDDEN_DIM), lambda i: (0, 0)),
            pl.BlockSpec((NUM_EXPERTS, HIDDEN_DIM), lambda i: (0, 0)),
            pl.BlockSpec((NUM_EXPERTS, FEATURE_DIM), lambda i: (0, 0)),
            pl.BlockSpec((1, NUM_EXPERTS), lambda i: (0, 0)),
            pl.BlockSpec((1, NUM_EXPERTS), lambda i: (0, 0)),
        ],
        out_specs=pl.BlockSpec((BLOCK_B, NUM_EXPERTS), lambda i: (i, 0)),
        out_shape=jax.ShapeDtypeStruct((N_TOKENS, NUM_EXPERTS), jnp.float32),
        compiler_params=pltpu.CompilerParams(
            dimension_semantics=("arbitrary",)),
    )(features, W_proj, b2, expert_emb, expert_features, tr2, st2)


# speedup vs baseline: 1.6585x; 1.1761x over previous
import jax
import jax.numpy as jnp
from jax.experimental import pallas as pl
from jax.experimental.pallas import tpu as pltpu

FEATURE_DIM = 1024
HIDDEN_DIM = 256
NUM_EXPERTS = 64
TOP_K = 8
STALENESS_LAMBDA = 0.005
STALENESS_FLOOR = 0.1
N_TOKENS = 16384

BLOCK_B = 1024


def _router_kernel(f_ref, w_ref, b_ref, emb_ref, ef_ref, tr_ref, st_ref,
                   out_ref):
    f = f_ref[:]
    fb = f.astype(jnp.bfloat16)
    h = jnp.dot(fb, w_ref[:].astype(jnp.bfloat16),
                preferred_element_type=jnp.float32) + b_ref[:]
    logits = jax.lax.dot_general(
        h.astype(jnp.bfloat16), emb_ref[:].astype(jnp.bfloat16),
        (((1,), (1,)), ((), ())),
        preferred_element_type=jnp.float32)
    m = jnp.max(logits, axis=-1, keepdims=True)
    p = jnp.exp(logits - m)
    gate = p / jnp.sum(p, axis=-1, keepdims=True)

    ef = ef_ref[:]
    en = ef / (jnp.sqrt(jnp.sum(ef * ef, axis=1, keepdims=True)) + 1e-8)
    rnorm = jnp.sqrt(jnp.sum(f * f, axis=1, keepdims=True)) + 1e-8
    fn = f / rnorm
    sim = jnp.maximum(jax.lax.dot_general(
        fn.astype(jnp.bfloat16), en.astype(jnp.bfloat16),
        (((1,), (1,)), ((), ())),
        preferred_element_type=jnp.float32), 0.0)

    scores = ((gate * tr_ref[:]) * sim) * st_ref[:]

    work = scores
    removed = jnp.zeros(scores.shape[:1] + (1,), jnp.float32)
    kth = jnp.zeros(scores.shape[:1] + (1,), jnp.float32)
    for _ in range(TOP_K):
        mx = jnp.max(work, axis=-1, keepdims=True)
        ge = work >= mx
        live = removed < float(TOP_K)
        kth = jnp.where(live, mx, kth)
        removed = removed + jnp.sum(
            jnp.where(ge, 1.0, 0.0), axis=-1, keepdims=True)
        work = jnp.where(ge, -1.0, work)

    masked = jnp.where(scores >= kth, scores, 0.0)
    out_ref[:] = masked / (jnp.sum(masked, axis=-1, keepdims=True) + 1e-9)


@jax.jit
def kernel(features, W_proj, b_proj, expert_emb, expert_features, trust,
           staleness_dt):
    stale = jnp.maximum(jnp.exp(-STALENESS_LAMBDA * staleness_dt),
                        STALENESS_FLOOR)
    tr2 = trust.reshape(1, NUM_EXPERTS)
    st2 = stale.reshape(1, NUM_EXPERTS)
    b2 = b_proj.reshape(1, HIDDEN_DIM)
    n_blocks = N_TOKENS // BLOCK_B
    return pl.pallas_call(
        _router_kernel,
        grid=(n_blocks,),
        in_specs=[
            pl.BlockSpec((BLOCK_B, FEATURE_DIM), lambda i: (i, 0)),
            pl.BlockSpec((FEATURE_DIM, HIDDEN_DIM), lambda i: (0, 0)),
            pl.BlockSpec((1, HIDDEN_DIM), lambda i: (0, 0)),
            pl.BlockSpec((NUM_EXPERTS, HIDDEN_DIM), lambda i: (0, 0)),
            pl.BlockSpec((NUM_EXPERTS, FEATURE_DIM), lambda i: (0, 0)),
            pl.BlockSpec((1, NUM_EXPERTS), lambda i: (0, 0)),
            pl.BlockSpec((1, NUM_EXPERTS), lambda i: (0, 0)),
        ],
        out_specs=pl.BlockSpec((BLOCK_B, NUM_EXPERTS), lambda i: (i, 0)),
        out_shape=jax.ShapeDtypeStruct((N_TOKENS, NUM_EXPERTS), jnp.float32),
        compiler_params=pltpu.CompilerParams(
            dimension_semantics=("arbitrary",)),
    )(features, W_proj, b2, expert_emb, expert_features, tr2, st2)


# speedup vs baseline: 1.6869x; 1.0171x over previous
import jax
import jax.numpy as jnp
from jax.experimental import pallas as pl
from jax.experimental.pallas import tpu as pltpu

FEATURE_DIM = 1024
HIDDEN_DIM = 256
NUM_EXPERTS = 64
TOP_K = 8
STALENESS_LAMBDA = 0.005
STALENESS_FLOOR = 0.1
N_TOKENS = 16384

BLOCK_B = 1024


def _router_kernel(f_ref, w_ref, b_ref, emb_ref, ef_ref, ts_ref, out_ref):
    f = f_ref[:]
    fb = f.astype(jnp.bfloat16)
    h = jnp.dot(fb, w_ref[:].astype(jnp.bfloat16),
                preferred_element_type=jnp.float32) + b_ref[:]
    logits = jax.lax.dot_general(
        h.astype(jnp.bfloat16), emb_ref[:].astype(jnp.bfloat16),
        (((1,), (1,)), ((), ())),
        preferred_element_type=jnp.float32)
    p = jnp.exp(logits)

    ef = ef_ref[:]
    en = ef / (jnp.sqrt(jnp.sum(ef * ef, axis=1, keepdims=True)) + 1e-8)
    rnorm = jnp.sqrt(jnp.sum(f * f, axis=1, keepdims=True)) + 1e-8
    fn = f / rnorm
    sim = jnp.maximum(jax.lax.dot_general(
        fn.astype(jnp.bfloat16), en.astype(jnp.bfloat16),
        (((1,), (1,)), ((), ())),
        preferred_element_type=jnp.float32), 0.0)

    scores = (p * sim) * ts_ref[:]

    work = scores
    removed = jnp.zeros(scores.shape[:1] + (1,), jnp.float32)
    kth = jnp.zeros(scores.shape[:1] + (1,), jnp.float32)
    for _ in range(TOP_K):
        mx = jnp.max(work, axis=-1, keepdims=True)
        ge = work >= mx
        live = removed < float(TOP_K)
        kth = jnp.where(live, mx, kth)
        removed = removed + jnp.sum(
            jnp.where(ge, 1.0, 0.0), axis=-1, keepdims=True)
        work = jnp.where(ge, -1.0, work)

    masked = jnp.where(scores >= kth, scores, 0.0)
    out_ref[:] = masked / (jnp.sum(masked, axis=-1, keepdims=True) + 1e-9)


@jax.jit
def kernel(features, W_proj, b_proj, expert_emb, expert_features, trust,
           staleness_dt):
    stale = jnp.maximum(jnp.exp(-STALENESS_LAMBDA * staleness_dt),
                        STALENESS_FLOOR)
    ts2 = (trust * stale).reshape(1, NUM_EXPERTS)
    b2 = b_proj.reshape(1, HIDDEN_DIM)
    n_blocks = N_TOKENS // BLOCK_B
    return pl.pallas_call(
        _router_kernel,
        grid=(n_blocks,),
        in_specs=[
            pl.BlockSpec((BLOCK_B, FEATURE_DIM), lambda i: (i, 0)),
            pl.BlockSpec((FEATURE_DIM, HIDDEN_DIM), lambda i: (0, 0)),
            pl.BlockSpec((1, HIDDEN_DIM), lambda i: (0, 0)),
            pl.BlockSpec((NUM_EXPERTS, HIDDEN_DIM), lambda i: (0, 0)),
            pl.BlockSpec((NUM_EXPERTS, FEATURE_DIM), lambda i: (0, 0)),
            pl.BlockSpec((1, NUM_EXPERTS), lambda i: (0, 0)),
        ],
        out_specs=pl.BlockSpec((BLOCK_B, NUM_EXPERTS), lambda i: (i, 0)),
        out_shape=jax.ShapeDtypeStruct((N_TOKENS, NUM_EXPERTS), jnp.float32),
        compiler_params=pltpu.CompilerParams(
            dimension_semantics=("arbitrary",)),
    )(features, W_proj, b2, expert_emb, expert_features, ts2)
